# baseline (speedup 1.0000x reference)
import functools

import jax
import jax.numpy as jnp
from jax.experimental import pallas as pl
from jax.experimental.pallas import tpu as pltpu


def _conv_kernel(x_ref, w_ref, shift_ref, o_ref, *, kh, kw):
    b, h, w, c_in = x_ref.shape
    for bi in range(b):
        xb = x_ref[bi].astype(jnp.bfloat16)
        zcol = jnp.zeros((h, 1, c_in), dtype=jnp.bfloat16)
        xw = jnp.concatenate([zcol, xb, zcol], axis=1)
        zrow = jnp.zeros((1, w + kw - 1, c_in), dtype=jnp.bfloat16)
        xp = jnp.concatenate([zrow, xw, zrow], axis=0)
        taps = [xp[i:i + h, j:j + w, :]
                for i in range(kh) for j in range(kw)]
        patches = jnp.concatenate(taps, axis=2)
        p2 = patches.reshape(h * w, kh * kw * c_in)
        acc = jnp.dot(p2, w_ref[...],
                      preferred_element_type=jnp.float32)
        acc = jnp.maximum(acc + shift_ref[...], 0.0)
        o_ref[bi] = acc.reshape(h, w, acc.shape[-1])


@jax.jit
def _basic_conv2d_opt(x_nchw, weight_oihw, gamma, beta, running_mean,
                      running_var):
    eps = 1e-3
    n, c_in, h, w = x_nchw.shape
    c_out, c_in_w, kh, kw = weight_oihw.shape
    assert c_in == c_in_w
    oh, ow = h, w
    k_dim = kh * kw * c_in
    blk = 4 if n % 4 == 0 else (2 if n % 2 == 0 else 1)

    x_nhwc = jnp.transpose(x_nchw, (0, 2, 3, 1))

    scale = gamma.astype(jnp.float32) / jnp.sqrt(
        running_var.astype(jnp.float32) + eps)
    shift = beta.astype(jnp.float32) - running_mean.astype(jnp.float32) * scale
    w_mat = jnp.transpose(weight_oihw, (2, 3, 1, 0)).reshape(k_dim, c_out)
    w_mat = (w_mat.astype(jnp.float32) * scale[None, :]).astype(jnp.bfloat16)
    shift_row = shift.reshape(1, c_out)

    out_nhwc = pl.pallas_call(
        functools.partial(_conv_kernel, kh=kh, kw=kw),
        out_shape=jax.ShapeDtypeStruct((n, oh, ow, c_out), jnp.float32),
        grid_spec=pltpu.PrefetchScalarGridSpec(
            num_scalar_prefetch=0,
            grid=(n // blk,),
            in_specs=[
                pl.BlockSpec((blk, h, w, c_in), lambda i: (i, 0, 0, 0)),
                pl.BlockSpec((k_dim, c_out), lambda i: (0, 0)),
                pl.BlockSpec((1, c_out), lambda i: (0, 0)),
            ],
            out_specs=pl.BlockSpec((blk, oh, ow, c_out),
                                   lambda i: (i, 0, 0, 0)),
        ),
        compiler_params=pltpu.CompilerParams(
            dimension_semantics=("parallel",),
            vmem_limit_bytes=64 * 1024 * 1024,
        ),
        cost_estimate=pl.CostEstimate(
            flops=2 * n * oh * ow * k_dim * c_out,
            transcendentals=0,
            bytes_accessed=n * (h * w * c_in * 4 + oh * ow * c_out * 4)
            + k_dim * c_out * 2,
        ),
    )(x_nhwc, w_mat, shift_row)

    return jnp.transpose(out_nhwc, (0, 3, 1, 2))


def kernel(x_nchw, weight_oihw, gamma, beta, running_mean, running_var):
    return _basic_conv2d_opt(x_nchw, weight_oihw, gamma, beta, running_mean,
                             running_var)
